# Initial kernel scaffold; baseline (speedup 1.0000x reference)
import jax
import jax.numpy as jnp
from jax.experimental import pallas as pl


def kernel(x):
    raise NotImplementedError("write your pallas kernel here")


# speedup vs baseline: 55.6403x; 55.6403x over previous
import jax
import jax.numpy as jnp
from jax.experimental import pallas as pl
from jax.experimental.pallas import tpu as pltpu


def _pad_rows_reflect2(a, h):
    return jnp.concatenate(
        [a[2:3], a[1:2], a, a[h - 2:h - 1], a[h - 3:h - 2]], axis=0)


def _pad_cols_reflect2(a, w):
    return jnp.concatenate(
        [a[:, 2:3], a[:, 1:2], a, a[:, w - 2:w - 1], a[:, w - 3:w - 2]],
        axis=1)


def kernel(x):
    b, s, c, h, w = x.shape
    gk = (0.0625, 0.25, 0.375, 0.25, 0.0625)
    sm = (1.0, 4.0, 6.0, 4.0, 1.0)

    def body(x_ref, o_ref, best_ref):
        si = pl.program_id(1)
        img = (x_ref[0, 0, 0] + x_ref[0, 0, 1] + x_ref[0, 0, 2]) * (1.0 / 3.0)
        img = img.astype(jnp.bfloat16).astype(jnp.float32)

        pr = _pad_rows_reflect2(img, h)
        t = gk[0] * pr[0:h]
        for d in range(1, 5):
            t = t + gk[d] * pr[d:d + h]
        pc = _pad_cols_reflect2(t, w)
        blur = gk[0] * pc[:, 0:w]
        for d in range(1, 5):
            blur = blur + gk[d] * pc[:, d:d + w]

        blur = blur.astype(jnp.bfloat16).astype(jnp.float32)
        pr2 = _pad_rows_reflect2(blur, h)
        a_rows = sm[0] * pr2[0:h]
        for d in range(1, 5):
            a_rows = a_rows + sm[d] * pr2[d:d + h]
        b_rows = pr2[0:h] - 2.0 * pr2[2:h + 2] + pr2[4:h + 4]
        pa = _pad_cols_reflect2(a_rows, w)
        pb = _pad_cols_reflect2(b_rows, w)
        lap = pa[:, 0:w] - 2.0 * pa[:, 2:w + 2] + pa[:, 4:w + 4]
        lap_b = sm[0] * pb[:, 0:w]
        for d in range(1, 5):
            lap_b = lap_b + sm[d] * pb[:, d:d + w]
        lap = lap + lap_b

        @pl.when(si == 0)
        def _init():
            best_ref[...] = lap
            for ci in range(c):
                o_ref[0, ci] = x_ref[0, 0, ci]

        @pl.when(si > 0)
        def _update():
            prev = best_ref[...]
            pred = lap > prev
            best_ref[...] = jnp.where(pred, lap, prev)
            for ci in range(c):
                o_ref[0, ci] = jnp.where(pred, x_ref[0, 0, ci], o_ref[0, ci])

    return pl.pallas_call(
        body,
        grid=(b, s),
        in_specs=[
            pl.BlockSpec((1, 1, c, h, w), lambda i, j: (i, j, 0, 0, 0)),
        ],
        out_specs=pl.BlockSpec((1, c, h, w), lambda i, j: (i, 0, 0, 0)),
        out_shape=jax.ShapeDtypeStruct((b, c, h, w), x.dtype),
        scratch_shapes=[pltpu.VMEM((h, w), jnp.float32)],
        compiler_params=pltpu.CompilerParams(
            dimension_semantics=("arbitrary", "arbitrary")),
    )(x)


# speedup vs baseline: 115.9114x; 2.0832x over previous
import numpy as np
import jax
import jax.numpy as jnp
from jax.experimental import pallas as pl
from jax.experimental.pallas import tpu as pltpu


def _banded_reflect_colmat(weights, n):
    m = np.zeros((n, n), np.float32)
    for j in range(n):
        for d, wt in enumerate(weights):
            if wt == 0.0:
                continue
            idx = j + d - 2
            if idx < 0:
                idx = -idx
            elif idx >= n:
                idx = 2 * n - 2 - idx
            m[idx, j] += wt
    return m


def _pad_rows_reflect2(a, h):
    return jnp.concatenate(
        [a[2:3], a[1:2], a, a[h - 2:h - 1], a[h - 3:h - 2]], axis=0)


def _row_conv(p, weights, h):
    out = None
    for d, wt in enumerate(weights):
        if wt == 0.0:
            continue
        term = p[d:d + h] if wt == 1.0 else wt * p[d:d + h]
        out = term if out is None else out + term
    return out


def kernel(x):
    b, s, c, h, w = x.shape
    gk = (0.0625, 0.25, 0.375, 0.25, 0.0625)
    sm = (1.0, 4.0, 6.0, 4.0, 1.0)
    d2 = (1.0, 0.0, -2.0, 0.0, 1.0)

    gmat = jnp.asarray(_banded_reflect_colmat(gk, w), jnp.bfloat16)
    lmat = jnp.asarray(
        np.concatenate([_banded_reflect_colmat(d2, w),
                        _banded_reflect_colmat(sm, w)], axis=1), jnp.bfloat16)

    def body(x_ref, g_ref, l_ref, o_ref, best_ref):
        si = pl.program_id(1)
        img = (x_ref[0, 0, 0] + x_ref[0, 0, 1] + x_ref[0, 0, 2]) * (1.0 / 3.0)
        imgb = img.astype(jnp.bfloat16)

        t = jax.lax.dot_general(imgb, g_ref[...], (((1,), (0,)), ((), ())),
                                preferred_element_type=jnp.float32)
        blur = _row_conv(_pad_rows_reflect2(t, h), gk, h)

        blurb = blur.astype(jnp.bfloat16)
        ab = jax.lax.dot_general(blurb, l_ref[...], (((1,), (0,)), ((), ())),
                                 preferred_element_type=jnp.float32)
        pa = _pad_rows_reflect2(ab[:, 0:w], h)
        pb = _pad_rows_reflect2(ab[:, w:2 * w], h)
        lap = _row_conv(pa, sm, h) + _row_conv(pb, d2, h)

        @pl.when(si == 0)
        def _init():
            best_ref[...] = lap
            for ci in range(c):
                o_ref[0, ci] = x_ref[0, 0, ci]

        @pl.when(si > 0)
        def _update():
            prev = best_ref[...]
            pred = lap > prev
            best_ref[...] = jnp.where(pred, lap, prev)
            for ci in range(c):
                o_ref[0, ci] = jnp.where(pred, x_ref[0, 0, ci], o_ref[0, ci])

    return pl.pallas_call(
        body,
        grid=(b, s),
        in_specs=[
            pl.BlockSpec((1, 1, c, h, w), lambda i, j: (i, j, 0, 0, 0)),
            pl.BlockSpec((w, w), lambda i, j: (0, 0)),
            pl.BlockSpec((w, 2 * w), lambda i, j: (0, 0)),
        ],
        out_specs=pl.BlockSpec((1, c, h, w), lambda i, j: (i, 0, 0, 0)),
        out_shape=jax.ShapeDtypeStruct((b, c, h, w), x.dtype),
        scratch_shapes=[pltpu.VMEM((h, w), jnp.float32)],
        compiler_params=pltpu.CompilerParams(
            dimension_semantics=("parallel", "arbitrary")),
    )(x, gmat, lmat)
